# baseline (speedup 1.0000x reference)
import jax
import jax.numpy as jnp
from jax.experimental import pallas as pl
from jax.experimental.pallas import tpu as pltpu

_N, _E, _G = 1024, 2048, 32
_NC = 32
_EBLK = 256


def _matmul(a, b):
    return jax.lax.dot_general(
        a, b, (((1,), (0,)), ((), ())), preferred_element_type=jnp.float32
    )


def _ntdot(a, b):
    return jax.lax.dot_general(
        a, b, (((1,), (1,)), ((), ())), preferred_element_type=jnp.float32
    )


def _tndot(a, b):
    return jax.lax.dot_general(
        a, b, (((0,), (0,)), ((), ())), preferred_element_type=jnp.float32
    )


def _gcn_body(X_ref, Z_ref, adj_e_hbm, adj_v_hbm, T_hbm, Pt_ref,
              W1_ref, p1_ref, b1_ref, W2_ref, p2t_ref, b2_ref,
              W3_ref, p3_ref, b3_ref, Wl_ref, bl_ref,
              out_ref, adj_e_ref, adj_v_ref, T_ref, zh2_s, sem_e, sem_v, sem_t):
    cp_t = pltpu.make_async_copy(T_hbm, T_ref, sem_t)
    cp_t.start()
    cp_v = pltpu.make_async_copy(adj_v_hbm, adj_v_ref, sem_v)
    cp_v.start()
    cp_e = pltpu.make_async_copy(adj_e_hbm, adj_e_ref, sem_e)
    cp_e.start()
    X = X_ref[...]
    Z = Z_ref[...]

    rows = jax.lax.broadcasted_iota(jnp.int32, (_N, _N), 0)
    cols = jax.lax.broadcasted_iota(jnp.int32, (_N, _N), 1)
    diag_v = rows == cols

    d1 = _ntdot(p1_ref[...], Z)
    Hw1 = _matmul(X, W1_ref[...])
    cp_t.wait()
    T = T_ref[...]
    Tb = T.astype(jnp.bfloat16)
    mult1 = _ntdot((T * d1).astype(jnp.bfloat16), Tb)
    cp_v.wait()
    A1 = jnp.where(diag_v, adj_v_ref[...], mult1 * adj_v_ref[...])
    Xh1 = jnp.maximum(
        _matmul(A1.astype(jnp.bfloat16), Hw1.astype(jnp.bfloat16))
        + b1_ref[...], 0.0)

    Zr = jnp.maximum(Z, 0.0)
    HeW = _matmul(Zr, W2_ref[...])
    d2 = _matmul(Xh1, p2t_ref[...])
    Ts = (d2 * T).astype(jnp.bfloat16)
    HeWb = HeW.astype(jnp.bfloat16)
    cp_e.wait()
    for k in range(_E // _EBLK):
        r0 = k * _EBLK
        mult_blk = _tndot(Ts[:, r0:r0 + _EBLK], Tb)
        adj_blk = adj_e_ref[pl.ds(r0, _EBLK), :]
        rr = jax.lax.broadcasted_iota(jnp.int32, (_EBLK, _E), 0)
        cc = jax.lax.broadcasted_iota(jnp.int32, (_EBLK, _E), 1)
        A_blk = jnp.where(cc == rr + r0, adj_blk, mult_blk * adj_blk)
        zh2_s[pl.ds(r0, _EBLK), :] = jnp.maximum(
            _matmul(A_blk.astype(jnp.bfloat16), HeWb) + b2_ref[...], 0.0)
    Zh2 = zh2_s[...]

    d3 = _ntdot(p3_ref[...], Zh2)
    Hw3 = _matmul(Xh1, W3_ref[...])
    mult3 = _ntdot((T * d3).astype(jnp.bfloat16), Tb)
    A3 = jnp.where(diag_v, adj_v_ref[...], mult3 * adj_v_ref[...])
    Xh3 = jnp.maximum(
        _matmul(A3.astype(jnp.bfloat16), Hw3.astype(jnp.bfloat16))
        + b3_ref[...], 0.0)

    Pt = Pt_ref[...]
    xh = Xh3.astype(jnp.bfloat16)
    xl = (Xh3 - xh.astype(jnp.float32)).astype(jnp.bfloat16)
    Ptb = Pt.astype(jnp.bfloat16)
    pooled = _matmul(Ptb, xh) + _matmul(Ptb, xl)
    counts = jnp.sum(Pt, axis=1, keepdims=True)
    mean = pooled / jnp.maximum(counts, 1.0)
    out_ref[...] = _matmul(mean, Wl_ref[...]) + bl_ref[...]


def kernel(X, Z, adj_e, adj_v, T, batch, W1, p1, b1, W2, p2, b2, W3, p3, b3, Wl, bl):
    Pt = (batch.astype(jnp.int32)[None, :]
          == jnp.arange(_G, dtype=jnp.int32)[:, None]).astype(jnp.float32)
    vm = pl.BlockSpec(memory_space=pltpu.MemorySpace.HBM)
    specs = [None, None, vm, vm, vm] + [None] * 12
    specs = [s if s is not None else pl.BlockSpec(memory_space=pltpu.MemorySpace.VMEM)
             for s in specs]
    return pl.pallas_call(
        _gcn_body,
        out_shape=jax.ShapeDtypeStruct((_G, _NC), jnp.float32),
        in_specs=specs,
        scratch_shapes=[pltpu.VMEM((_E, _E), jnp.float32),
                        pltpu.VMEM((_N, _N), jnp.float32),
                        pltpu.VMEM((_N, _E), jnp.float32),
                        pltpu.VMEM((_E, 16), jnp.float32),
                        pltpu.SemaphoreType.DMA,
                        pltpu.SemaphoreType.DMA,
                        pltpu.SemaphoreType.DMA],
    )(X, Z, adj_e, adj_v, T, Pt,
      W1, p1, b1.reshape(1, -1), W2, p2.reshape(-1, 1), b2.reshape(1, -1),
      W3, p3, b3.reshape(1, -1), Wl, bl.reshape(1, -1))


# speedup vs baseline: 1.0894x; 1.0894x over previous
import jax
import jax.numpy as jnp
from jax.experimental import pallas as pl
from jax.experimental.pallas import tpu as pltpu

_N, _E, _G = 1024, 2048, 32
_NC = 32
_EBLK = 256


def _matmul(a, b):
    return jax.lax.dot_general(
        a, b, (((1,), (0,)), ((), ())), preferred_element_type=jnp.float32
    )


def _ntdot(a, b):
    return jax.lax.dot_general(
        a, b, (((1,), (1,)), ((), ())), preferred_element_type=jnp.float32
    )


def _tndot(a, b):
    return jax.lax.dot_general(
        a, b, (((0,), (0,)), ((), ())), preferred_element_type=jnp.float32
    )


def _gcn_body(X_ref, Z_ref, adj_e_hbm, adj_v_hbm, T_ref, batch_ref,
              W1_ref, p1_ref, b1_ref, W2_ref, p2_ref, b2_ref,
              W3_ref, p3_ref, b3_ref, Wl_ref, bl_ref,
              out_ref, adj_e_ref, adj_v_ref, zh2_s, sem_e, sem_v):
    cp_v = pltpu.make_async_copy(adj_v_hbm, adj_v_ref, sem_v)
    cp_v.start()
    cp_e = pltpu.make_async_copy(adj_e_hbm, adj_e_ref, sem_e)
    cp_e.start()
    X = X_ref[...]
    Z = Z_ref[...]
    T = T_ref[...]

    rows = jax.lax.broadcasted_iota(jnp.int32, (_N, _N), 0)
    cols = jax.lax.broadcasted_iota(jnp.int32, (_N, _N), 1)
    diag_v = rows == cols

    gi = jax.lax.broadcasted_iota(jnp.int32, (_G, _N), 0)
    Pt = (batch_ref[...] == gi).astype(jnp.float32)

    d1 = _ntdot(p1_ref[...], Z)
    Hw1 = _matmul(X, W1_ref[...])
    mult1 = _ntdot(T * d1, T)
    cp_v.wait()
    A1 = jnp.where(diag_v, adj_v_ref[...], mult1 * adj_v_ref[...])
    Xh1 = jnp.maximum(_matmul(A1, Hw1) + b1_ref[...], 0.0)

    Zr = jnp.maximum(Z, 0.0)
    HeW = _matmul(Zr, W2_ref[...])
    d2 = _ntdot(Xh1, p2_ref[...])
    Ts = d2 * T
    cp_e.wait()
    for k in range(_E // _EBLK):
        r0 = k * _EBLK
        mult_blk = _tndot(Ts[:, r0:r0 + _EBLK], T)
        adj_blk = adj_e_ref[pl.ds(r0, _EBLK), :]
        rr = jax.lax.broadcasted_iota(jnp.int32, (_EBLK, _E), 0)
        cc = jax.lax.broadcasted_iota(jnp.int32, (_EBLK, _E), 1)
        A_blk = jnp.where(cc == rr + r0, adj_blk, mult_blk * adj_blk)
        zh2_s[pl.ds(r0, _EBLK), :] = jnp.maximum(
            _matmul(A_blk, HeW) + b2_ref[...], 0.0)
    Zh2 = zh2_s[...]

    d3 = _ntdot(p3_ref[...], Zh2)
    Hw3 = _matmul(Xh1, W3_ref[...])
    mult3 = _ntdot(T * d3, T)
    A3 = jnp.where(diag_v, adj_v_ref[...], mult3 * adj_v_ref[...])
    Xh3 = jnp.maximum(_matmul(A3, Hw3) + b3_ref[...], 0.0)

    xh = Xh3.astype(jnp.bfloat16)
    xl = (Xh3 - xh.astype(jnp.float32)).astype(jnp.bfloat16)
    pooled = (_matmul(Pt, xh.astype(jnp.float32))
              + _matmul(Pt, xl.astype(jnp.float32)))
    counts = jnp.sum(Pt, axis=1, keepdims=True)
    mean = pooled / jnp.maximum(counts, 1.0)
    out_ref[...] = _matmul(mean, Wl_ref[...]) + bl_ref[...]


def kernel(X, Z, adj_e, adj_v, T, batch, W1, p1, b1, W2, p2, b2, W3, p3, b3, Wl, bl):
    hbm = pl.BlockSpec(memory_space=pltpu.MemorySpace.HBM)
    vmem = pl.BlockSpec(memory_space=pltpu.MemorySpace.VMEM)
    specs = [vmem, vmem, hbm, hbm] + [vmem] * 13
    return pl.pallas_call(
        _gcn_body,
        out_shape=jax.ShapeDtypeStruct((_G, _NC), jnp.float32),
        in_specs=specs,
        scratch_shapes=[pltpu.VMEM((_E, _E), jnp.float32),
                        pltpu.VMEM((_N, _N), jnp.float32),
                        pltpu.VMEM((_E, 16), jnp.float32),
                        pltpu.SemaphoreType.DMA,
                        pltpu.SemaphoreType.DMA],
    )(X, Z, adj_e, adj_v, T, batch.astype(jnp.int32).reshape(1, _N),
      W1, p1, b1.reshape(1, -1), W2, p2, b2.reshape(1, -1),
      W3, p3, b3.reshape(1, -1), Wl, bl.reshape(1, -1))
